# Initial kernel scaffold; baseline (speedup 1.0000x reference)
import jax
import jax.numpy as jnp
from jax.experimental import pallas as pl


def kernel(xyz1, xyz2, feat1, feat2, W0, b0, g0, be0, W1, b1, g1, be1):
    raise NotImplementedError("write your pallas kernel here")


# speedup vs baseline: 10.3450x; 10.3450x over previous
import functools

import jax
import jax.numpy as jnp
from jax import lax
from jax.experimental import pallas as pl
from jax.experimental.pallas import tpu as pltpu
from jax.experimental.pallas import tpu_sc as plsc

_B = 2
_N = 2048
_M = 2048
_K = 16
_C1 = 64
_C2 = 64
_CO = 64
_D = 128
_TN = 256
_TN2 = 256
_TP = _TN2 * _K
_NK = _N * _K
_EPS = 1e-5


def _topk_body(x1_ref, x2_ref, idx_ref):
    b = pl.program_id(0)
    x1 = x1_ref[0]
    x2 = x2_ref[0]
    acc = jnp.zeros((_TN, _M), jnp.float32)
    for c in range(3):
        dcol = x1[:, c:c + 1] - x2[c:c + 1, :]
        acc = acc + dcol * dcol
    d = jnp.sqrt(acc)
    iota = lax.broadcasted_iota(jnp.int32, (_TN, _M), 1)
    cols = []
    for _ in range(_K):
        mn = jnp.min(d, axis=1, keepdims=True)
        am = jnp.min(jnp.where(d == mn, iota, _M), axis=1, keepdims=True)
        cols.append(am)
        d = jnp.where(iota == am, jnp.inf, d)
    idx = jnp.concatenate(cols, axis=1)
    idx_ref[0] = idx + b * _M


def _topk(xyz1_p, xyz2_p):
    grid = (_B, _N // _TN)
    return pl.pallas_call(
        _topk_body,
        grid=grid,
        in_specs=[
            pl.BlockSpec((1, _TN, 8), lambda b, t: (b, t, 0)),
            pl.BlockSpec((1, 8, _M), lambda b, t: (b, 0, 0)),
        ],
        out_specs=pl.BlockSpec((1, _TN, _K), lambda b, t: (b, t, 0)),
        out_shape=jax.ShapeDtypeStruct((_B, _N, _K), jnp.int32),
    )(xyz1_p, xyz2_p)


def _sc_gather(table, idx3):
    info = plsc.get_sparse_core_info()
    nw = info.num_cores * info.num_subcores
    btot = _B * _NK
    b_per_w = btot // nw
    ch_sz = 128
    n_ch = b_per_w // ch_sz
    mesh = plsc.VectorSubcoreMesh(core_axis_name="c", subcore_axis_name="s")

    @functools.partial(
        pl.kernel,
        mesh=mesh,
        out_type=jax.ShapeDtypeStruct((btot, _D), jnp.float32),
        scratch_types=[
            pltpu.VMEM((n_ch, ch_sz), jnp.int32),
            pltpu.VMEM((ch_sz, _D), jnp.float32),
            pltpu.SemaphoreType.DMA,
        ],
    )
    def gk(table_hbm, idx_hbm, out_hbm, idx_v, rows_v, sem):
        wid = lax.axis_index("s") * info.num_cores + lax.axis_index("c")
        base = wid * b_per_w
        pltpu.sync_copy(idx_hbm.at[wid], idx_v)
        for ch in range(n_ch):
            pltpu.async_copy(table_hbm.at[idx_v.at[ch]], rows_v, sem).wait()
            pltpu.sync_copy(rows_v, out_hbm.at[pl.ds(base + ch * ch_sz, ch_sz)])

    return gk(table, idx3)


def _leaky(x):
    return jnp.where(x >= 0, x, 0.1 * x)


def _mlp1_body(rows_ref, f1_ref, w0r_ref, w0f_ref, b0_ref, h0_ref, st_ref):
    t = pl.program_id(1)
    g = rows_ref[0]
    f1 = f1_ref[0]
    hr = jnp.dot(g, w0r_ref[...], preferred_element_type=jnp.float32)
    hf = jnp.dot(f1, w0f_ref[...], preferred_element_type=jnp.float32)
    h = hr.reshape(_TN2, _K, _CO) + hf[:, None, :] + b0_ref[...][None]
    h2 = h.reshape(_TP, _CO)
    h0_ref[0] = h2
    s = jnp.sum(h2, axis=0, keepdims=True)
    q = jnp.sum(h2 * h2, axis=0, keepdims=True)
    st = jnp.concatenate([s, q, jnp.zeros((6, _CO), jnp.float32)], axis=0)

    @pl.when(t == 0)
    def _():
        st_ref[0] = st

    @pl.when(t > 0)
    def _():
        st_ref[0] += st


def _mlp1(rows, f1aug, w0r, w0f, b0r):
    grid = (_B, _N // _TN2)
    return pl.pallas_call(
        _mlp1_body,
        grid=grid,
        in_specs=[
            pl.BlockSpec((1, _TP, _D), lambda b, t: (b, t, 0)),
            pl.BlockSpec((1, _TN2, _D), lambda b, t: (b, t, 0)),
            pl.BlockSpec((_D, _CO), lambda b, t: (0, 0)),
            pl.BlockSpec((_D, _CO), lambda b, t: (0, 0)),
            pl.BlockSpec((1, _CO), lambda b, t: (0, 0)),
        ],
        out_specs=[
            pl.BlockSpec((1, _TP, _CO), lambda b, t: (b, t, 0)),
            pl.BlockSpec((1, 8, _CO), lambda b, t: (b, 0, 0)),
        ],
        out_shape=[
            jax.ShapeDtypeStruct((_B, _NK, _CO), jnp.float32),
            jax.ShapeDtypeStruct((_B, 8, _CO), jnp.float32),
        ],
    )(rows, f1aug, w0r, w0f, b0r)


def _mlp2_body(h0_ref, sc_ref, w1_ref, b1_ref, h1_ref, st_ref):
    t = pl.program_id(1)
    h = h0_ref[0]
    scale = sc_ref[0, 0:1, :]
    shift = sc_ref[0, 1:2, :]
    a = _leaky(h * scale + shift)
    h1 = jnp.dot(a, w1_ref[...], preferred_element_type=jnp.float32) + b1_ref[...]
    h1_ref[0] = h1
    s = jnp.sum(h1, axis=0, keepdims=True)
    q = jnp.sum(h1 * h1, axis=0, keepdims=True)
    st = jnp.concatenate([s, q, jnp.zeros((6, _CO), jnp.float32)], axis=0)

    @pl.when(t == 0)
    def _():
        st_ref[0] = st

    @pl.when(t > 0)
    def _():
        st_ref[0] += st


def _mlp2(h0, sc0, w1t, b1r):
    grid = (_B, _N // _TN2)
    return pl.pallas_call(
        _mlp2_body,
        grid=grid,
        in_specs=[
            pl.BlockSpec((1, _TP, _CO), lambda b, t: (b, t, 0)),
            pl.BlockSpec((1, 8, _CO), lambda b, t: (b, 0, 0)),
            pl.BlockSpec((_CO, _CO), lambda b, t: (0, 0)),
            pl.BlockSpec((1, _CO), lambda b, t: (0, 0)),
        ],
        out_specs=[
            pl.BlockSpec((1, _TP, _CO), lambda b, t: (b, t, 0)),
            pl.BlockSpec((1, 8, _CO), lambda b, t: (b, 0, 0)),
        ],
        out_shape=[
            jax.ShapeDtypeStruct((_B, _NK, _CO), jnp.float32),
            jax.ShapeDtypeStruct((_B, 8, _CO), jnp.float32),
        ],
    )(h0, sc0, w1t, b1r)


def _mlp3_body(h1_ref, sc_ref, out_ref):
    h = h1_ref[0]
    scale = sc_ref[0, 0:1, :]
    shift = sc_ref[0, 1:2, :]
    a = _leaky(h * scale + shift)
    a3 = a.reshape(_TN2, _K, _CO)
    out_ref[0] = jnp.max(a3, axis=1)


def _mlp3(h1, sc1):
    grid = (_B, _N // _TN2)
    return pl.pallas_call(
        _mlp3_body,
        grid=grid,
        in_specs=[
            pl.BlockSpec((1, _TP, _CO), lambda b, t: (b, t, 0)),
            pl.BlockSpec((1, 8, _CO), lambda b, t: (b, 0, 0)),
        ],
        out_specs=pl.BlockSpec((1, _TN2, _CO), lambda b, t: (b, t, 0)),
        out_shape=jax.ShapeDtypeStruct((_B, _N, _CO), jnp.float32),
    )(h1, sc1)


def _fold_stats(st, gamma, beta, count):
    s = st[:, 0, :]
    q = st[:, 1, :]
    groups = _CO // 16
    sg = s.reshape(_B, groups, 16).sum(-1)
    qg = q.reshape(_B, groups, 16).sum(-1)
    mean = sg / count
    var = qg / count - mean * mean
    inv = 1.0 / jnp.sqrt(var + _EPS)
    inv_c = jnp.repeat(inv, 16, axis=1)
    mean_c = jnp.repeat(mean, 16, axis=1)
    scale = gamma[None, :] * inv_c
    shift = beta[None, :] - mean_c * scale
    pad = jnp.zeros((_B, 6, _CO), jnp.float32)
    return jnp.concatenate([scale[:, None, :], shift[:, None, :], pad], axis=1)


def kernel(xyz1, xyz2, feat1, feat2, W0, b0, g0, be0, W1, b1, g1, be1):
    xyz1_p = jnp.concatenate(
        [jnp.transpose(xyz1, (0, 2, 1)),
         jnp.zeros((_B, _N, 5), jnp.float32)], axis=2)
    xyz2_p = jnp.concatenate(
        [xyz2, jnp.zeros((_B, 5, _M), jnp.float32)], axis=1)
    table = jnp.concatenate(
        [jnp.transpose(feat2, (0, 2, 1)),
         jnp.transpose(xyz2, (0, 2, 1)),
         jnp.zeros((_B, _M, _D - _C2 - 3), jnp.float32)],
        axis=2).reshape(_B * _M, _D)
    f1aug = jnp.concatenate(
        [jnp.transpose(feat1, (0, 2, 1)),
         jnp.transpose(xyz1, (0, 2, 1)),
         jnp.zeros((_B, _N, _D - _C1 - 3), jnp.float32)],
        axis=2)
    w0r = jnp.zeros((_D, _CO), jnp.float32)
    w0r = w0r.at[: _C2, :].set(W0[:, _C1:_C1 + _C2].T)
    w0r = w0r.at[_C2:_C2 + 3, :].set(W0[:, _C1 + _C2:].T)
    w0f = jnp.zeros((_D, _CO), jnp.float32)
    w0f = w0f.at[: _C1, :].set(W0[:, : _C1].T)
    w0f = w0f.at[_C1:_C1 + 3, :].set(-W0[:, _C1 + _C2:].T)
    b0r = b0.reshape(1, _CO)
    w1t = W1.T
    b1r = b1.reshape(1, _CO)

    fidx = _topk(xyz1_p, xyz2_p)
    idx3 = fidx.reshape(32, (_B * _NK) // (32 * 128), 128)

    rows = _sc_gather(table, idx3).reshape(_B, _NK, _D)

    count = jnp.float32(16 * _N * _K)
    h0, st0 = _mlp1(rows, f1aug, w0r, w0f, b0r)
    sc0 = _fold_stats(st0, g0, be0, count)
    h1, st1 = _mlp2(h0, sc0, w1t, b1r)
    sc1 = _fold_stats(st1, g1, be1, count)
    out_t = _mlp3(h1, sc1)
    return jnp.transpose(out_t, (0, 2, 1))


# speedup vs baseline: 11.9435x; 1.1545x over previous
import functools

import jax
import jax.numpy as jnp
from jax import lax
from jax.experimental import pallas as pl
from jax.experimental.pallas import tpu as pltpu
from jax.experimental.pallas import tpu_sc as plsc

_B = 2
_N = 2048
_M = 2048
_K = 16
_C1 = 64
_C2 = 64
_CO = 64
_D = 128
_TN = 256
_TN2 = 256
_TP = _TN2 * _K
_NK = _N * _K
_EPS = 1e-5
_QB = (1 << 20) - 1


def _topk_body(x1t_ref, x1_ref, x2_ref, x2t_ref, f2_ref, f1_ref,
               wfa_ref, wfb_ref, b0_ref, idx_ref, tab_ref, hf_ref):
    b = pl.program_id(0)
    t = pl.program_id(1)

    @pl.when(t == 0)
    def _():
        tab_ref[0, :, 0:_C2] = f2_ref[0].T
        tab_ref[0, :, _C2:_C2 + 3] = x2t_ref[0]
        tab_ref[0, :, _C2 + 3:] = jnp.zeros((_M, _D - _C2 - 3), jnp.float32)

    hf = (
        jnp.dot(wfa_ref[...], f1_ref[0], preferred_element_type=jnp.float32)
        + jnp.dot(wfb_ref[...], x1_ref[0], preferred_element_type=jnp.float32)
        + b0_ref[...]
    )
    hf_ref[0] = hf

    x1t = x1t_ref[0]
    x2 = x2_ref[0]
    d0 = x1t[:, 0:1] - x2[0:1, :]
    d1 = x1t[:, 1:2] - x2[1:2, :]
    d2c = x1t[:, 2:3] - x2[2:3, :]
    d2 = d0 * d0 + d1 * d1 + d2c * d2c

    cm = jnp.min(d2.reshape(_TN, 16, 128), axis=2)
    hi = jnp.max(cm, axis=1, keepdims=True)
    scale = jnp.float32(_QB) / jnp.maximum(hi, jnp.float32(1e-37))
    dq = jnp.minimum(d2 * scale, jnp.float32(_QB))
    di = dq.astype(jnp.int32)
    iota = lax.broadcasted_iota(jnp.int32, (_TN, _M), 1)
    keys = (di << 11) | iota

    cols = []
    for _ in range(_K):
        mn = jnp.min(keys, axis=1, keepdims=True)
        cols.append(mn)
        keys = jnp.where(keys == mn, jnp.int32(0x7FFFFFFF), keys)
    idx = jnp.concatenate(cols, axis=1) & jnp.int32(0x7FF)
    idx_ref[0] = idx + b * _M


def _topk(xyz1t, xyz1, xyz2, xyz2t, feat2, feat1, wfa, wfb, b0c):
    grid = (_B, _N // _TN)
    return pl.pallas_call(
        _topk_body,
        grid=grid,
        in_specs=[
            pl.BlockSpec((1, _TN, 3), lambda b, t: (b, t, 0)),
            pl.BlockSpec((1, 8, _TN), lambda b, t: (b, 0, t)),
            pl.BlockSpec((1, 3, _M), lambda b, t: (b, 0, 0)),
            pl.BlockSpec((1, _M, 3), lambda b, t: (b, 0, 0)),
            pl.BlockSpec((1, _C2, _M), lambda b, t: (b, 0, 0)),
            pl.BlockSpec((1, _C1, _TN), lambda b, t: (b, 0, t)),
            pl.BlockSpec((_CO, _C1), lambda b, t: (0, 0)),
            pl.BlockSpec((_CO, 8), lambda b, t: (0, 0)),
            pl.BlockSpec((_CO, 1), lambda b, t: (0, 0)),
        ],
        out_specs=[
            pl.BlockSpec((1, _TN, _K), lambda b, t: (b, t, 0)),
            pl.BlockSpec((1, _M, _D), lambda b, t: (b, 0, 0)),
            pl.BlockSpec((1, _CO, _TN), lambda b, t: (b, 0, t)),
        ],
        out_shape=[
            jax.ShapeDtypeStruct((_B, _N, _K), jnp.int32),
            jax.ShapeDtypeStruct((_B, _M, _D), jnp.float32),
            jax.ShapeDtypeStruct((_B, _CO, _N), jnp.float32),
        ],
    )(xyz1t, xyz1, xyz2, xyz2t, feat2, feat1, wfa, wfb, b0c)


def _sc_gather(table, idx3):
    info = plsc.get_sparse_core_info()
    nw = info.num_cores * info.num_subcores
    btot = _B * _NK
    b_per_w = btot // nw
    ch_sz = 128
    n_ch = b_per_w // ch_sz
    mesh = plsc.VectorSubcoreMesh(core_axis_name="c", subcore_axis_name="s")

    @functools.partial(
        pl.kernel,
        mesh=mesh,
        out_type=jax.ShapeDtypeStruct((btot, _D), jnp.float32),
        scratch_types=[
            pltpu.VMEM((n_ch, ch_sz), jnp.int32),
            pltpu.VMEM((ch_sz, _D), jnp.float32),
            pltpu.SemaphoreType.DMA,
        ],
    )
    def gk(table_hbm, idx_hbm, out_hbm, idx_v, rows_v, sem):
        wid = lax.axis_index("s") * info.num_cores + lax.axis_index("c")
        base = wid * b_per_w
        pltpu.sync_copy(idx_hbm.at[wid], idx_v)
        for ch in range(n_ch):
            pltpu.async_copy(table_hbm.at[idx_v.at[ch]], rows_v, sem).wait()
            pltpu.sync_copy(rows_v, out_hbm.at[pl.ds(base + ch * ch_sz, ch_sz)])

    return gk(table, idx3)


def _leaky(x):
    return jnp.where(x >= 0, x, 0.1 * x)


def _fold(st_ref, g_ref, be_ref):
    sq = st_ref[0, 0:2, :]
    ri = lax.broadcasted_iota(jnp.int32, (_CO, _CO), 0) >> 4
    ci = lax.broadcasted_iota(jnp.int32, (_CO, _CO), 1) >> 4
    G = (ri == ci).astype(jnp.float32)
    sqg = jnp.dot(sq, G, preferred_element_type=jnp.float32)
    count = jnp.float32(16 * _N * _K)
    mean = sqg[0:1, :] / count
    var = sqg[1:2, :] / count - mean * mean
    inv = 1.0 / jnp.sqrt(var + _EPS)
    scale = g_ref[...] * inv
    shift = be_ref[...] - mean * scale
    return scale, shift


def _mlp1_body(rows_ref, hf_ref, w0r_ref, h0_ref, st_ref):
    t = pl.program_id(1)
    g = rows_ref[0]
    hr = jnp.dot(g, w0r_ref[...], preferred_element_type=jnp.float32)
    hf = hf_ref[0].T
    h = hr.reshape(_TN2, _K, _CO) + hf[:, None, :]
    h2 = h.reshape(_TP, _CO)
    h0_ref[0] = h2
    s = jnp.sum(h2, axis=0, keepdims=True)
    q = jnp.sum(h2 * h2, axis=0, keepdims=True)
    st = jnp.concatenate([s, q, jnp.zeros((6, _CO), jnp.float32)], axis=0)

    @pl.when(t == 0)
    def _():
        st_ref[0] = st

    @pl.when(t > 0)
    def _():
        st_ref[0] += st


def _mlp1(rows, hft, w0r):
    grid = (_B, _N // _TN2)
    return pl.pallas_call(
        _mlp1_body,
        grid=grid,
        in_specs=[
            pl.BlockSpec((1, _TP, _D), lambda b, t: (b, t, 0)),
            pl.BlockSpec((1, _CO, _TN2), lambda b, t: (b, 0, t)),
            pl.BlockSpec((_D, _CO), lambda b, t: (0, 0)),
        ],
        out_specs=[
            pl.BlockSpec((1, _TP, _CO), lambda b, t: (b, t, 0)),
            pl.BlockSpec((1, 8, _CO), lambda b, t: (b, 0, 0)),
        ],
        out_shape=[
            jax.ShapeDtypeStruct((_B, _NK, _CO), jnp.float32),
            jax.ShapeDtypeStruct((_B, 8, _CO), jnp.float32),
        ],
    )(rows, hft, w0r)


def _mlp2_body(h0_ref, st_ref, g_ref, be_ref, w1_ref, b1_ref, h1_ref, so_ref):
    t = pl.program_id(1)
    scale, shift = _fold(st_ref, g_ref, be_ref)
    h = h0_ref[0]
    a = _leaky(h * scale + shift)
    h1 = jnp.dot(a, w1_ref[...], preferred_element_type=jnp.float32) + b1_ref[...]
    h1_ref[0] = h1
    s = jnp.sum(h1, axis=0, keepdims=True)
    q = jnp.sum(h1 * h1, axis=0, keepdims=True)
    st = jnp.concatenate([s, q, jnp.zeros((6, _CO), jnp.float32)], axis=0)

    @pl.when(t == 0)
    def _():
        so_ref[0] = st

    @pl.when(t > 0)
    def _():
        so_ref[0] += st


def _mlp2(h0, st0, g0, be0, w1t, b1r):
    grid = (_B, _N // _TN2)
    return pl.pallas_call(
        _mlp2_body,
        grid=grid,
        in_specs=[
            pl.BlockSpec((1, _TP, _CO), lambda b, t: (b, t, 0)),
            pl.BlockSpec((1, 8, _CO), lambda b, t: (b, 0, 0)),
            pl.BlockSpec((1, _CO), lambda b, t: (0, 0)),
            pl.BlockSpec((1, _CO), lambda b, t: (0, 0)),
            pl.BlockSpec((_CO, _CO), lambda b, t: (0, 0)),
            pl.BlockSpec((1, _CO), lambda b, t: (0, 0)),
        ],
        out_specs=[
            pl.BlockSpec((1, _TP, _CO), lambda b, t: (b, t, 0)),
            pl.BlockSpec((1, 8, _CO), lambda b, t: (b, 0, 0)),
        ],
        out_shape=[
            jax.ShapeDtypeStruct((_B, _NK, _CO), jnp.float32),
            jax.ShapeDtypeStruct((_B, 8, _CO), jnp.float32),
        ],
    )(h0, st0, g0, be0, w1t, b1r)


def _mlp3_body(h1_ref, st_ref, g_ref, be_ref, out_ref):
    scale, shift = _fold(st_ref, g_ref, be_ref)
    h = h1_ref[0]
    a = _leaky(h * scale + shift)
    a3 = a.reshape(_TN2, _K, _CO)
    m = jnp.max(a3, axis=1)
    out_ref[0] = m.T


def _mlp3(h1, st1, g1, be1):
    grid = (_B, _N // _TN2)
    return pl.pallas_call(
        _mlp3_body,
        grid=grid,
        in_specs=[
            pl.BlockSpec((1, _TP, _CO), lambda b, t: (b, t, 0)),
            pl.BlockSpec((1, 8, _CO), lambda b, t: (b, 0, 0)),
            pl.BlockSpec((1, _CO), lambda b, t: (0, 0)),
            pl.BlockSpec((1, _CO), lambda b, t: (0, 0)),
        ],
        out_specs=pl.BlockSpec((1, _CO, _TN2), lambda b, t: (b, 0, t)),
        out_shape=jax.ShapeDtypeStruct((_B, _CO, _N), jnp.float32),
    )(h1, st1, g1, be1)


def kernel(xyz1, xyz2, feat1, feat2, W0, b0, g0, be0, W1, b1, g1, be1):
    xyz1t = jnp.transpose(xyz1, (0, 2, 1))
    xyz1p = jnp.concatenate(
        [xyz1, jnp.zeros((_B, 5, _N), jnp.float32)], axis=1)
    xyz2t = jnp.transpose(xyz2, (0, 2, 1))
    w0r = jnp.zeros((_D, _CO), jnp.float32)
    w0r = w0r.at[: _C2, :].set(W0[:, _C1:_C1 + _C2].T)
    w0r = w0r.at[_C2:_C2 + 3, :].set(W0[:, _C1 + _C2:].T)
    wfa = W0[:, : _C1]
    wfb = jnp.zeros((_CO, 8), jnp.float32)
    wfb = wfb.at[:, :3].set(-W0[:, _C1 + _C2:])
    b0c = b0.reshape(_CO, 1)
    w1t = W1.T
    b1r = b1.reshape(1, _CO)
    g0r = g0.reshape(1, _CO)
    be0r = be0.reshape(1, _CO)
    g1r = g1.reshape(1, _CO)
    be1r = be1.reshape(1, _CO)

    fidx, tab, hft = _topk(xyz1t, xyz1p, xyz2, xyz2t, feat2, feat1,
                           wfa, wfb, b0c)
    table = tab.reshape(_B * _M, _D)
    idx3 = fidx.reshape(32, (_B * _NK) // (32 * 128), 128)

    rows = _sc_gather(table, idx3).reshape(_B, _NK, _D)

    h0, st0 = _mlp1(rows, hft, w0r)
    h1, st1 = _mlp2(h0, st0, g0r, be0r, w1t, b1r)
    return _mlp3(h1, st1, g1r, be1r)


# speedup vs baseline: 11.9602x; 1.0014x over previous
import functools

import jax
import jax.numpy as jnp
from jax import lax
from jax.experimental import pallas as pl
from jax.experimental.pallas import tpu as pltpu
from jax.experimental.pallas import tpu_sc as plsc

_B = 2
_N = 2048
_M = 2048
_K = 16
_C1 = 64
_C2 = 64
_CO = 64
_D = 128
_TN = 256
_TN2 = 256
_TP = _TN2 * _K
_NK = _N * _K
_EPS = 1e-5
_QB = (1 << 20) - 1
_CP = pltpu.CompilerParams(dimension_semantics=("parallel", "arbitrary"))


def _topk_body(x1t_ref, x1_ref, x2_ref, x2t_ref, f2_ref, f1_ref,
               wfa_ref, wfb_ref, b0_ref, idx_ref, tab_ref, hf_ref):
    b = pl.program_id(0)
    t = pl.program_id(1)

    @pl.when(t == 0)
    def _():
        tab_ref[0, :, 0:_C2] = f2_ref[0].T
        tab_ref[0, :, _C2:_C2 + 3] = x2t_ref[0]
        tab_ref[0, :, _C2 + 3:] = jnp.zeros((_M, _D - _C2 - 3), jnp.float32)

    hf = (
        jnp.dot(wfa_ref[...], f1_ref[0], preferred_element_type=jnp.float32)
        + jnp.dot(wfb_ref[...], x1_ref[0], preferred_element_type=jnp.float32)
        + b0_ref[...]
    )
    hf_ref[0] = hf

    x1t = x1t_ref[0]
    x2 = x2_ref[0]
    d0 = x1t[:, 0:1] - x2[0:1, :]
    d1 = x1t[:, 1:2] - x2[1:2, :]
    d2c = x1t[:, 2:3] - x2[2:3, :]
    d2 = d0 * d0 + d1 * d1 + d2c * d2c

    cm = jnp.min(d2.reshape(_TN, 16, 128), axis=2)
    hi = jnp.max(cm, axis=1, keepdims=True)
    scale = jnp.float32(_QB) / jnp.maximum(hi, jnp.float32(1e-37))
    dq = jnp.minimum(d2 * scale, jnp.float32(_QB))
    di = dq.astype(jnp.int32)
    iota = lax.broadcasted_iota(jnp.int32, (_TN, _M), 1)
    keys = (di << 11) | iota

    cols = []
    for _ in range(_K):
        mn = jnp.min(keys, axis=1, keepdims=True)
        cols.append(mn)
        keys = jnp.where(keys == mn, jnp.int32(0x7FFFFFFF), keys)
    idx = jnp.concatenate(cols, axis=1) & jnp.int32(0x7FF)
    idx_ref[0] = idx + b * _M


def _topk(xyz1t, xyz1, xyz2, xyz2t, feat2, feat1, wfa, wfb, b0c):
    grid = (_B, _N // _TN)
    return pl.pallas_call(
        _topk_body,
        grid=grid,
        in_specs=[
            pl.BlockSpec((1, _TN, 3), lambda b, t: (b, t, 0)),
            pl.BlockSpec((1, 8, _TN), lambda b, t: (b, 0, t)),
            pl.BlockSpec((1, 3, _M), lambda b, t: (b, 0, 0)),
            pl.BlockSpec((1, _M, 3), lambda b, t: (b, 0, 0)),
            pl.BlockSpec((1, _C2, _M), lambda b, t: (b, 0, 0)),
            pl.BlockSpec((1, _C1, _TN), lambda b, t: (b, 0, t)),
            pl.BlockSpec((_CO, _C1), lambda b, t: (0, 0)),
            pl.BlockSpec((_CO, 8), lambda b, t: (0, 0)),
            pl.BlockSpec((_CO, 1), lambda b, t: (0, 0)),
        ],
        out_specs=[
            pl.BlockSpec((1, _TN, _K), lambda b, t: (b, t, 0)),
            pl.BlockSpec((1, _M, _D), lambda b, t: (b, 0, 0)),
            pl.BlockSpec((1, _CO, _TN), lambda b, t: (b, 0, t)),
        ],
        out_shape=[
            jax.ShapeDtypeStruct((_B, _N, _K), jnp.int32),
            jax.ShapeDtypeStruct((_B, _M, _D), jnp.float32),
            jax.ShapeDtypeStruct((_B, _CO, _N), jnp.float32),
        ],
        compiler_params=_CP,
    )(xyz1t, xyz1, xyz2, xyz2t, feat2, feat1, wfa, wfb, b0c)


def _sc_gather(table, idx3):
    info = plsc.get_sparse_core_info()
    nw = info.num_cores * info.num_subcores
    btot = _B * _NK
    b_per_w = btot // nw
    ch_sz = 128
    n_ch = b_per_w // ch_sz
    mesh = plsc.VectorSubcoreMesh(core_axis_name="c", subcore_axis_name="s")

    @functools.partial(
        pl.kernel,
        mesh=mesh,
        out_type=jax.ShapeDtypeStruct((btot, _D), jnp.float32),
        scratch_types=[
            pltpu.VMEM((n_ch, ch_sz), jnp.int32),
            pltpu.VMEM((ch_sz, _D), jnp.float32),
            pltpu.SemaphoreType.DMA,
        ],
    )
    def gk(table_hbm, idx_hbm, out_hbm, idx_v, rows_v, sem):
        wid = lax.axis_index("s") * info.num_cores + lax.axis_index("c")
        base = wid * b_per_w
        pltpu.sync_copy(idx_hbm.at[wid], idx_v)
        for ch in range(n_ch):
            pltpu.async_copy(table_hbm.at[idx_v.at[ch]], rows_v, sem).wait()
            pltpu.sync_copy(rows_v, out_hbm.at[pl.ds(base + ch * ch_sz, ch_sz)])

    return gk(table, idx3)


def _leaky(x):
    return jnp.where(x >= 0, x, 0.1 * x)


def _fold(st_ref, g_ref, be_ref):
    sq = st_ref[0, 0:2, :]
    ri = lax.broadcasted_iota(jnp.int32, (_CO, _CO), 0) >> 4
    ci = lax.broadcasted_iota(jnp.int32, (_CO, _CO), 1) >> 4
    G = (ri == ci).astype(jnp.float32)
    sqg = jnp.dot(sq, G, preferred_element_type=jnp.float32)
    count = jnp.float32(16 * _N * _K)
    mean = sqg[0:1, :] / count
    var = sqg[1:2, :] / count - mean * mean
    inv = 1.0 / jnp.sqrt(var + _EPS)
    scale = g_ref[...] * inv
    shift = be_ref[...] - mean * scale
    return scale, shift


def _mlp1_body(rows_ref, hf_ref, w0r_ref, h0_ref, st_ref):
    t = pl.program_id(1)
    g = rows_ref[0]
    hr = jnp.dot(g, w0r_ref[...], preferred_element_type=jnp.float32)
    hf = hf_ref[0].T
    h = hr.reshape(_TN2, _K, _CO) + hf[:, None, :]
    h2 = h.reshape(_TP, _CO)
    h0_ref[0] = h2
    s = jnp.sum(h2, axis=0, keepdims=True)
    q = jnp.sum(h2 * h2, axis=0, keepdims=True)
    st = jnp.concatenate([s, q, jnp.zeros((6, _CO), jnp.float32)], axis=0)

    @pl.when(t == 0)
    def _():
        st_ref[0] = st

    @pl.when(t > 0)
    def _():
        st_ref[0] += st


def _mlp1(rows, hft, w0r):
    grid = (_B, _N // _TN2)
    return pl.pallas_call(
        _mlp1_body,
        grid=grid,
        in_specs=[
            pl.BlockSpec((1, _TP, _D), lambda b, t: (b, t, 0)),
            pl.BlockSpec((1, _CO, _TN2), lambda b, t: (b, 0, t)),
            pl.BlockSpec((_D, _CO), lambda b, t: (0, 0)),
        ],
        out_specs=[
            pl.BlockSpec((1, _TP, _CO), lambda b, t: (b, t, 0)),
            pl.BlockSpec((1, 8, _CO), lambda b, t: (b, 0, 0)),
        ],
        out_shape=[
            jax.ShapeDtypeStruct((_B, _NK, _CO), jnp.float32),
            jax.ShapeDtypeStruct((_B, 8, _CO), jnp.float32),
        ],
        compiler_params=_CP,
    )(rows, hft, w0r)


def _mlp2_body(h0_ref, st_ref, g_ref, be_ref, w1_ref, b1_ref, h1_ref, so_ref):
    t = pl.program_id(1)
    scale, shift = _fold(st_ref, g_ref, be_ref)
    h = h0_ref[0]
    a = _leaky(h * scale + shift)
    h1 = jnp.dot(a, w1_ref[...], preferred_element_type=jnp.float32) + b1_ref[...]
    h1_ref[0] = h1
    s = jnp.sum(h1, axis=0, keepdims=True)
    q = jnp.sum(h1 * h1, axis=0, keepdims=True)
    st = jnp.concatenate([s, q, jnp.zeros((6, _CO), jnp.float32)], axis=0)

    @pl.when(t == 0)
    def _():
        so_ref[0] = st

    @pl.when(t > 0)
    def _():
        so_ref[0] += st


def _mlp2(h0, st0, g0, be0, w1t, b1r):
    grid = (_B, _N // _TN2)
    return pl.pallas_call(
        _mlp2_body,
        grid=grid,
        in_specs=[
            pl.BlockSpec((1, _TP, _CO), lambda b, t: (b, t, 0)),
            pl.BlockSpec((1, 8, _CO), lambda b, t: (b, 0, 0)),
            pl.BlockSpec((1, _CO), lambda b, t: (0, 0)),
            pl.BlockSpec((1, _CO), lambda b, t: (0, 0)),
            pl.BlockSpec((_CO, _CO), lambda b, t: (0, 0)),
            pl.BlockSpec((1, _CO), lambda b, t: (0, 0)),
        ],
        out_specs=[
            pl.BlockSpec((1, _TP, _CO), lambda b, t: (b, t, 0)),
            pl.BlockSpec((1, 8, _CO), lambda b, t: (b, 0, 0)),
        ],
        out_shape=[
            jax.ShapeDtypeStruct((_B, _NK, _CO), jnp.float32),
            jax.ShapeDtypeStruct((_B, 8, _CO), jnp.float32),
        ],
        compiler_params=_CP,
    )(h0, st0, g0, be0, w1t, b1r)


def _mlp3_body(h1_ref, st_ref, g_ref, be_ref, out_ref):
    scale, shift = _fold(st_ref, g_ref, be_ref)
    h = h1_ref[0]
    a = _leaky(h * scale + shift)
    a3 = a.reshape(_TN2, _K, _CO)
    m = jnp.max(a3, axis=1)
    out_ref[0] = m.T


def _mlp3(h1, st1, g1, be1):
    grid = (_B, _N // _TN2)
    return pl.pallas_call(
        _mlp3_body,
        grid=grid,
        in_specs=[
            pl.BlockSpec((1, _TP, _CO), lambda b, t: (b, t, 0)),
            pl.BlockSpec((1, 8, _CO), lambda b, t: (b, 0, 0)),
            pl.BlockSpec((1, _CO), lambda b, t: (0, 0)),
            pl.BlockSpec((1, _CO), lambda b, t: (0, 0)),
        ],
        out_specs=pl.BlockSpec((1, _CO, _TN2), lambda b, t: (b, 0, t)),
        out_shape=jax.ShapeDtypeStruct((_B, _CO, _N), jnp.float32),
        compiler_params=_CP,
    )(h1, st1, g1, be1)


def kernel(xyz1, xyz2, feat1, feat2, W0, b0, g0, be0, W1, b1, g1, be1):
    xyz1t = jnp.transpose(xyz1, (0, 2, 1))
    xyz1p = jnp.concatenate(
        [xyz1, jnp.zeros((_B, 5, _N), jnp.float32)], axis=1)
    xyz2t = jnp.transpose(xyz2, (0, 2, 1))
    w0r = jnp.zeros((_D, _CO), jnp.float32)
    w0r = w0r.at[: _C2, :].set(W0[:, _C1:_C1 + _C2].T)
    w0r = w0r.at[_C2:_C2 + 3, :].set(W0[:, _C1 + _C2:].T)
    wfa = W0[:, : _C1]
    wfb = jnp.zeros((_CO, 8), jnp.float32)
    wfb = wfb.at[:, :3].set(-W0[:, _C1 + _C2:])
    b0c = b0.reshape(_CO, 1)
    w1t = W1.T
    b1r = b1.reshape(1, _CO)
    g0r = g0.reshape(1, _CO)
    be0r = be0.reshape(1, _CO)
    g1r = g1.reshape(1, _CO)
    be1r = be1.reshape(1, _CO)

    fidx, tab, hft = _topk(xyz1t, xyz1p, xyz2, xyz2t, feat2, feat1,
                           wfa, wfb, b0c)
    table = tab.reshape(_B * _M, _D)
    idx3 = fidx.reshape(32, (_B * _NK) // (32 * 128), 128)

    rows = _sc_gather(table, idx3).reshape(_B, _NK, _D)

    h0, st0 = _mlp1(rows, hft, w0r)
    h1, st1 = _mlp2(h0, st0, g0r, be0r, w1t, b1r)
    return _mlp3(h1, st1, g1r, be1r)


# speedup vs baseline: 13.8355x; 1.1568x over previous
import functools

import jax
import jax.numpy as jnp
from jax import lax
from jax.experimental import pallas as pl
from jax.experimental.pallas import tpu as pltpu
from jax.experimental.pallas import tpu_sc as plsc

_B = 2
_N = 2048
_M = 2048
_K = 16
_C1 = 64
_C2 = 64
_CO = 64
_D = 128
_TN = 256
_TN2 = 256
_TP = _TN2 * _K
_NK = _N * _K
_EPS = 1e-5
_QB = (1 << 19) - 1
_CP = pltpu.CompilerParams(dimension_semantics=("parallel", "arbitrary"))


def _topk_body(x1t_ref, x1_ref, x2_ref, x2t_ref, f2_ref, f1_ref,
               wfa_ref, wfb_ref, b0_ref, idx_ref, tab_ref, hf_ref):
    b = pl.program_id(0)
    t = pl.program_id(1)

    @pl.when(t == 0)
    def _():
        tab_ref[0, :, 0:_C2] = f2_ref[0].T
        tab_ref[0, :, _C2:_C2 + 3] = x2t_ref[0]
        tab_ref[0, :, _C2 + 3:] = jnp.zeros((_M, _D - _C2 - 3), jnp.float32)

    hf = (
        jnp.dot(wfa_ref[...], f1_ref[0], preferred_element_type=jnp.float32)
        + jnp.dot(wfb_ref[...], x1_ref[0], preferred_element_type=jnp.float32)
        + b0_ref[...]
    )
    hf_ref[0] = hf

    x1t = x1t_ref[0]
    x2 = x2_ref[0]
    d0 = x1t[:, 0:1] - x2[0:1, :]
    d1 = x1t[:, 1:2] - x2[1:2, :]
    d2c = x1t[:, 2:3] - x2[2:3, :]
    d2 = d0 * d0 + d1 * d1 + d2c * d2c

    cm = jnp.min(d2.reshape(_TN, 16, 128), axis=2)
    hi = jnp.max(cm, axis=1, keepdims=True)
    scale = jnp.float32(_QB) / jnp.maximum(hi, jnp.float32(1e-37))
    dq = jnp.minimum(d2 * scale, jnp.float32(_QB))
    di = dq.astype(jnp.int32)
    iota = lax.broadcasted_iota(jnp.int32, (_TN, _M), 1)
    keys = lax.bitcast_convert_type(
        ((di << 11) | iota) + jnp.int32(0x00800000), jnp.float32)

    big = lax.bitcast_convert_type(jnp.int32(0x7F000000), jnp.float32)
    cols = []
    for _ in range(_K):
        mn = jnp.min(keys, axis=1, keepdims=True)
        cols.append(mn)
        keys = jnp.where(keys == mn, big, keys)
    idx = lax.bitcast_convert_type(
        jnp.concatenate(cols, axis=1), jnp.int32) & jnp.int32(0x7FF)
    idx_ref[0] = idx + b * _M


def _topk(xyz1t, xyz1, xyz2, xyz2t, feat2, feat1, wfa, wfb, b0c):
    grid = (_B, _N // _TN)
    return pl.pallas_call(
        _topk_body,
        grid=grid,
        in_specs=[
            pl.BlockSpec((1, _TN, 3), lambda b, t: (b, t, 0)),
            pl.BlockSpec((1, 8, _TN), lambda b, t: (b, 0, t)),
            pl.BlockSpec((1, 3, _M), lambda b, t: (b, 0, 0)),
            pl.BlockSpec((1, _M, 3), lambda b, t: (b, 0, 0)),
            pl.BlockSpec((1, _C2, _M), lambda b, t: (b, 0, 0)),
            pl.BlockSpec((1, _C1, _TN), lambda b, t: (b, 0, t)),
            pl.BlockSpec((_CO, _C1), lambda b, t: (0, 0)),
            pl.BlockSpec((_CO, 8), lambda b, t: (0, 0)),
            pl.BlockSpec((_CO, 1), lambda b, t: (0, 0)),
        ],
        out_specs=[
            pl.BlockSpec((1, _TN, _K), lambda b, t: (b, t, 0)),
            pl.BlockSpec((1, _M, _D), lambda b, t: (b, 0, 0)),
            pl.BlockSpec((1, _CO, _TN), lambda b, t: (b, 0, t)),
        ],
        out_shape=[
            jax.ShapeDtypeStruct((_B, _N, _K), jnp.int32),
            jax.ShapeDtypeStruct((_B, _M, _D), jnp.float32),
            jax.ShapeDtypeStruct((_B, _CO, _N), jnp.float32),
        ],
        compiler_params=_CP,
    )(xyz1t, xyz1, xyz2, xyz2t, feat2, feat1, wfa, wfb, b0c)


def _sc_gather(table, idx3):
    info = plsc.get_sparse_core_info()
    nw = info.num_cores * info.num_subcores
    btot = _B * _NK
    b_per_w = btot // nw
    ch_sz = 128
    n_ch = b_per_w // ch_sz
    mesh = plsc.VectorSubcoreMesh(core_axis_name="c", subcore_axis_name="s")

    @functools.partial(
        pl.kernel,
        mesh=mesh,
        out_type=jax.ShapeDtypeStruct((btot, _D), jnp.float32),
        scratch_types=[
            pltpu.VMEM((n_ch, ch_sz), jnp.int32),
            pltpu.VMEM((ch_sz, _D), jnp.float32),
            pltpu.SemaphoreType.DMA,
        ],
    )
    def gk(table_hbm, idx_hbm, out_hbm, idx_v, rows_v, sem):
        wid = lax.axis_index("s") * info.num_cores + lax.axis_index("c")
        base = wid * b_per_w
        pltpu.sync_copy(idx_hbm.at[wid], idx_v)
        for ch in range(n_ch):
            pltpu.async_copy(table_hbm.at[idx_v.at[ch]], rows_v, sem).wait()
            pltpu.sync_copy(rows_v, out_hbm.at[pl.ds(base + ch * ch_sz, ch_sz)])

    return gk(table, idx3)


def _leaky(x):
    return jnp.where(x >= 0, x, 0.1 * x)


def _fold(st_ref, g_ref, be_ref):
    sq = st_ref[0, 0:2, :]
    ri = lax.broadcasted_iota(jnp.int32, (_CO, _CO), 0) >> 4
    ci = lax.broadcasted_iota(jnp.int32, (_CO, _CO), 1) >> 4
    G = (ri == ci).astype(jnp.float32)
    sqg = jnp.dot(sq, G, preferred_element_type=jnp.float32)
    count = jnp.float32(16 * _N * _K)
    mean = sqg[0:1, :] / count
    var = sqg[1:2, :] / count - mean * mean
    inv = 1.0 / jnp.sqrt(var + _EPS)
    scale = g_ref[...] * inv
    shift = be_ref[...] - mean * scale
    return scale, shift


def _mlp1_body(rows_ref, hf_ref, w0r_ref, h0_ref, st_ref):
    t = pl.program_id(1)
    g = rows_ref[0]
    hr = jnp.dot(g, w0r_ref[...], preferred_element_type=jnp.float32)
    hf = hf_ref[0].T
    h = hr.reshape(_TN2, _K, _CO) + hf[:, None, :]
    h2 = h.reshape(_TP, _CO)
    h0_ref[0] = h2
    s = jnp.sum(h2, axis=0, keepdims=True)
    q = jnp.sum(h2 * h2, axis=0, keepdims=True)
    st = jnp.concatenate([s, q, jnp.zeros((6, _CO), jnp.float32)], axis=0)

    @pl.when(t == 0)
    def _():
        st_ref[0] = st

    @pl.when(t > 0)
    def _():
        st_ref[0] += st


def _mlp1(rows, hft, w0r):
    grid = (_B, _N // _TN2)
    return pl.pallas_call(
        _mlp1_body,
        grid=grid,
        in_specs=[
            pl.BlockSpec((1, _TP, _D), lambda b, t: (b, t, 0)),
            pl.BlockSpec((1, _CO, _TN2), lambda b, t: (b, 0, t)),
            pl.BlockSpec((_D, _CO), lambda b, t: (0, 0)),
        ],
        out_specs=[
            pl.BlockSpec((1, _TP, _CO), lambda b, t: (b, t, 0)),
            pl.BlockSpec((1, 8, _CO), lambda b, t: (b, 0, 0)),
        ],
        out_shape=[
            jax.ShapeDtypeStruct((_B, _NK, _CO), jnp.float32),
            jax.ShapeDtypeStruct((_B, 8, _CO), jnp.float32),
        ],
        compiler_params=_CP,
    )(rows, hft, w0r)


def _mlp2_body(h0_ref, st_ref, g_ref, be_ref, w1_ref, b1_ref, h1_ref, so_ref):
    t = pl.program_id(1)
    scale, shift = _fold(st_ref, g_ref, be_ref)
    h = h0_ref[0]
    a = _leaky(h * scale + shift)
    h1 = jnp.dot(a, w1_ref[...], preferred_element_type=jnp.float32) + b1_ref[...]
    h1_ref[0] = h1
    s = jnp.sum(h1, axis=0, keepdims=True)
    q = jnp.sum(h1 * h1, axis=0, keepdims=True)
    st = jnp.concatenate([s, q, jnp.zeros((6, _CO), jnp.float32)], axis=0)

    @pl.when(t == 0)
    def _():
        so_ref[0] = st

    @pl.when(t > 0)
    def _():
        so_ref[0] += st


def _mlp2(h0, st0, g0, be0, w1t, b1r):
    grid = (_B, _N // _TN2)
    return pl.pallas_call(
        _mlp2_body,
        grid=grid,
        in_specs=[
            pl.BlockSpec((1, _TP, _CO), lambda b, t: (b, t, 0)),
            pl.BlockSpec((1, 8, _CO), lambda b, t: (b, 0, 0)),
            pl.BlockSpec((1, _CO), lambda b, t: (0, 0)),
            pl.BlockSpec((1, _CO), lambda b, t: (0, 0)),
            pl.BlockSpec((_CO, _CO), lambda b, t: (0, 0)),
            pl.BlockSpec((1, _CO), lambda b, t: (0, 0)),
        ],
        out_specs=[
            pl.BlockSpec((1, _TP, _CO), lambda b, t: (b, t, 0)),
            pl.BlockSpec((1, 8, _CO), lambda b, t: (b, 0, 0)),
        ],
        out_shape=[
            jax.ShapeDtypeStruct((_B, _NK, _CO), jnp.float32),
            jax.ShapeDtypeStruct((_B, 8, _CO), jnp.float32),
        ],
        compiler_params=_CP,
    )(h0, st0, g0, be0, w1t, b1r)


def _mlp3_body(h1_ref, st_ref, g_ref, be_ref, out_ref):
    scale, shift = _fold(st_ref, g_ref, be_ref)
    h = h1_ref[0]
    a = _leaky(h * scale + shift)
    a3 = a.reshape(_TN2, _K, _CO)
    m = jnp.max(a3, axis=1)
    out_ref[0] = m.T


def _mlp3(h1, st1, g1, be1):
    grid = (_B, _N // _TN2)
    return pl.pallas_call(
        _mlp3_body,
        grid=grid,
        in_specs=[
            pl.BlockSpec((1, _TP, _CO), lambda b, t: (b, t, 0)),
            pl.BlockSpec((1, 8, _CO), lambda b, t: (b, 0, 0)),
            pl.BlockSpec((1, _CO), lambda b, t: (0, 0)),
            pl.BlockSpec((1, _CO), lambda b, t: (0, 0)),
        ],
        out_specs=pl.BlockSpec((1, _CO, _TN2), lambda b, t: (b, 0, t)),
        out_shape=jax.ShapeDtypeStruct((_B, _CO, _N), jnp.float32),
        compiler_params=_CP,
    )(h1, st1, g1, be1)


def kernel(xyz1, xyz2, feat1, feat2, W0, b0, g0, be0, W1, b1, g1, be1):
    xyz1t = jnp.transpose(xyz1, (0, 2, 1))
    xyz1p = jnp.concatenate(
        [xyz1, jnp.zeros((_B, 5, _N), jnp.float32)], axis=1)
    xyz2t = jnp.transpose(xyz2, (0, 2, 1))
    w0r = jnp.zeros((_D, _CO), jnp.float32)
    w0r = w0r.at[: _C2, :].set(W0[:, _C1:_C1 + _C2].T)
    w0r = w0r.at[_C2:_C2 + 3, :].set(W0[:, _C1 + _C2:].T)
    wfa = W0[:, : _C1]
    wfb = jnp.zeros((_CO, 8), jnp.float32)
    wfb = wfb.at[:, :3].set(-W0[:, _C1 + _C2:])
    b0c = b0.reshape(_CO, 1)
    w1t = W1.T
    b1r = b1.reshape(1, _CO)
    g0r = g0.reshape(1, _CO)
    be0r = be0.reshape(1, _CO)
    g1r = g1.reshape(1, _CO)
    be1r = be1.reshape(1, _CO)

    fidx, tab, hft = _topk(xyz1t, xyz1p, xyz2, xyz2t, feat2, feat1,
                           wfa, wfb, b0c)
    table = tab.reshape(_B * _M, _D)
    idx3 = fidx.reshape(32, (_B * _NK) // (32 * 128), 128)

    rows = _sc_gather(table, idx3).reshape(_B, _NK, _D)

    h0, st0 = _mlp1(rows, hft, w0r)
    h1, st1 = _mlp2(h0, st0, g0r, be0r, w1t, b1r)
    return _mlp3(h1, st1, g1r, be1r)


# speedup vs baseline: 14.3636x; 1.0382x over previous
import functools

import jax
import jax.numpy as jnp
from jax import lax
from jax.experimental import pallas as pl
from jax.experimental.pallas import tpu as pltpu
from jax.experimental.pallas import tpu_sc as plsc

_B = 2
_N = 2048
_M = 2048
_K = 16
_C1 = 64
_C2 = 64
_CO = 64
_D = 128
_DO = 64
_TN = 256
_TN2 = 256
_TP = _TN2 * _K
_NK = _N * _K
_EPS = 1e-5
_QB = (1 << 19) - 1
_CP = pltpu.CompilerParams(dimension_semantics=("parallel", "arbitrary"))


def _topk_body(x1t_ref, x1_ref, x2_ref, f2_ref, f1_ref,
               wfa_ref, wfb_ref, w2f_ref, w2x_ref, b0_ref,
               idx_ref, tab_ref, hf_ref):
    b = pl.program_id(0)
    t = pl.program_id(1)

    @pl.when(t == 0)
    def _():
        u = (
            jnp.dot(w2f_ref[...], f2_ref[0], preferred_element_type=jnp.float32)
            + jnp.dot(w2x_ref[...], x2_ref[0], preferred_element_type=jnp.float32)
        )
        tab_ref[0, :, 0:_DO] = u.T
        tab_ref[0, :, _DO:] = jnp.zeros((_M, _D - _DO), jnp.float32)

    hf = (
        jnp.dot(wfa_ref[...], f1_ref[0], preferred_element_type=jnp.float32)
        + jnp.dot(wfb_ref[...], x1_ref[0], preferred_element_type=jnp.float32)
        + b0_ref[...]
    )
    hf_ref[0] = hf

    x1t = x1t_ref[0]
    x2 = x2_ref[0]
    d0 = x1t[:, 0:1] - x2[0:1, :]
    d1 = x1t[:, 1:2] - x2[1:2, :]
    d2c = x1t[:, 2:3] - x2[2:3, :]
    d2 = d0 * d0 + d1 * d1 + d2c * d2c

    cm = jnp.min(d2.reshape(_TN, 16, 128), axis=2)
    hi = jnp.max(cm, axis=1, keepdims=True)
    scale = jnp.float32(_QB) / jnp.maximum(hi, jnp.float32(1e-37))
    dq = jnp.minimum(d2 * scale, jnp.float32(_QB))
    di = dq.astype(jnp.int32)
    iota = lax.broadcasted_iota(jnp.int32, (_TN, _M), 1)
    keys = lax.bitcast_convert_type(
        ((di << 11) | iota) + jnp.int32(0x00800000), jnp.float32)

    big = lax.bitcast_convert_type(jnp.int32(0x7F000000), jnp.float32)
    cols = []
    for _ in range(_K):
        mn = jnp.min(keys, axis=1, keepdims=True)
        cols.append(mn)
        keys = jnp.where(keys == mn, big, keys)
    idx = lax.bitcast_convert_type(
        jnp.concatenate(cols, axis=1), jnp.int32) & jnp.int32(0x7FF)
    idx_ref[0] = idx + b * _M


def _topk(xyz1t, xyz1, xyz2p, feat2, feat1, wfa, wfb, w2f, w2x, b0c):
    grid = (_B, _N // _TN)
    return pl.pallas_call(
        _topk_body,
        grid=grid,
        in_specs=[
            pl.BlockSpec((1, _TN, 3), lambda b, t: (b, t, 0)),
            pl.BlockSpec((1, 8, _TN), lambda b, t: (b, 0, t)),
            pl.BlockSpec((1, 8, _M), lambda b, t: (b, 0, 0)),
            pl.BlockSpec((1, _C2, _M), lambda b, t: (b, 0, 0)),
            pl.BlockSpec((1, _C1, _TN), lambda b, t: (b, 0, t)),
            pl.BlockSpec((_CO, _C1), lambda b, t: (0, 0)),
            pl.BlockSpec((_CO, 8), lambda b, t: (0, 0)),
            pl.BlockSpec((_CO, _C2), lambda b, t: (0, 0)),
            pl.BlockSpec((_CO, 8), lambda b, t: (0, 0)),
            pl.BlockSpec((_CO, 1), lambda b, t: (0, 0)),
        ],
        out_specs=[
            pl.BlockSpec((1, _TN, _K), lambda b, t: (b, t, 0)),
            pl.BlockSpec((1, _M, _D), lambda b, t: (b, 0, 0)),
            pl.BlockSpec((1, _CO, _TN), lambda b, t: (b, 0, t)),
        ],
        out_shape=[
            jax.ShapeDtypeStruct((_B, _N, _K), jnp.int32),
            jax.ShapeDtypeStruct((_B, _M, _D), jnp.float32),
            jax.ShapeDtypeStruct((_B, _CO, _N), jnp.float32),
        ],
        compiler_params=_CP,
    )(xyz1t, xyz1, xyz2p, feat2, feat1, wfa, wfb, w2f, w2x, b0c)


def _sc_gather(table, idx3):
    info = plsc.get_sparse_core_info()
    nw = info.num_cores * info.num_subcores
    btot = _B * _NK
    b_per_w = btot // nw
    ch_sz = 128
    n_ch = b_per_w // ch_sz
    mesh = plsc.VectorSubcoreMesh(core_axis_name="c", subcore_axis_name="s")

    @functools.partial(
        pl.kernel,
        mesh=mesh,
        out_type=jax.ShapeDtypeStruct((btot, _D), jnp.float32),
        scratch_types=[
            pltpu.VMEM((n_ch, ch_sz), jnp.int32),
            pltpu.VMEM((ch_sz, _D), jnp.float32),
            pltpu.SemaphoreType.DMA,
        ],
    )
    def gk(table_hbm, idx_hbm, out_hbm, idx_v, rows_v, sem):
        wid = lax.axis_index("s") * info.num_cores + lax.axis_index("c")
        base = wid * b_per_w
        pltpu.sync_copy(idx_hbm.at[wid], idx_v)
        for ch in range(n_ch):
            pltpu.async_copy(table_hbm.at[idx_v.at[ch]], rows_v, sem).wait()
            pltpu.sync_copy(rows_v, out_hbm.at[pl.ds(base + ch * ch_sz, ch_sz)])

    return gk(table, idx3)


def _leaky(x):
    return jnp.where(x >= 0, x, 0.1 * x)


def _fold(st_ref, g_ref, be_ref):
    sq = st_ref[0, 0:2, :]
    ri = lax.broadcasted_iota(jnp.int32, (_CO, _CO), 0) >> 4
    ci = lax.broadcasted_iota(jnp.int32, (_CO, _CO), 1) >> 4
    G = (ri == ci).astype(jnp.float32)
    sqg = jnp.dot(sq, G, preferred_element_type=jnp.float32)
    count = jnp.float32(16 * _N * _K)
    mean = sqg[0:1, :] / count
    var = sqg[1:2, :] / count - mean * mean
    inv = 1.0 / jnp.sqrt(var + _EPS)
    scale = g_ref[...] * inv
    shift = be_ref[...] - mean * scale
    return scale, shift


def _h0_tile(rows_ref, hf_ref):
    g = rows_ref[0][:, 0:_DO]
    hf = hf_ref[0].T
    h = g.reshape(_TN2, _K, _CO) + hf[:, None, :]
    return h.reshape(_TP, _CO)


def _mlp1_body(rows_ref, hf_ref, st_ref):
    t = pl.program_id(1)
    h2 = _h0_tile(rows_ref, hf_ref)
    s = jnp.sum(h2, axis=0, keepdims=True)
    q = jnp.sum(h2 * h2, axis=0, keepdims=True)
    st = jnp.concatenate([s, q, jnp.zeros((6, _CO), jnp.float32)], axis=0)

    @pl.when(t == 0)
    def _():
        st_ref[0] = st

    @pl.when(t > 0)
    def _():
        st_ref[0] += st


def _mlp1(rows, hft):
    grid = (_B, _N // _TN2)
    return pl.pallas_call(
        _mlp1_body,
        grid=grid,
        in_specs=[
            pl.BlockSpec((1, _TP, _D), lambda b, t: (b, t, 0)),
            pl.BlockSpec((1, _CO, _TN2), lambda b, t: (b, 0, t)),
        ],
        out_specs=pl.BlockSpec((1, 8, _CO), lambda b, t: (b, 0, 0)),
        out_shape=jax.ShapeDtypeStruct((_B, 8, _CO), jnp.float32),
        compiler_params=_CP,
    )(rows, hft)


def _mlp2_body(rows_ref, hf_ref, st_ref, g_ref, be_ref, w1_ref, b1_ref,
               h1_ref, so_ref):
    t = pl.program_id(1)
    scale, shift = _fold(st_ref, g_ref, be_ref)
    h = _h0_tile(rows_ref, hf_ref)
    a = _leaky(h * scale + shift)
    h1 = jnp.dot(a, w1_ref[...], preferred_element_type=jnp.float32) + b1_ref[...]
    h1_ref[0] = h1
    s = jnp.sum(h1, axis=0, keepdims=True)
    q = jnp.sum(h1 * h1, axis=0, keepdims=True)
    st = jnp.concatenate([s, q, jnp.zeros((6, _CO), jnp.float32)], axis=0)

    @pl.when(t == 0)
    def _():
        so_ref[0] = st

    @pl.when(t > 0)
    def _():
        so_ref[0] += st


def _mlp2(rows, hft, st0, g0, be0, w1t, b1r):
    grid = (_B, _N // _TN2)
    return pl.pallas_call(
        _mlp2_body,
        grid=grid,
        in_specs=[
            pl.BlockSpec((1, _TP, _D), lambda b, t: (b, t, 0)),
            pl.BlockSpec((1, _CO, _TN2), lambda b, t: (b, 0, t)),
            pl.BlockSpec((1, 8, _CO), lambda b, t: (b, 0, 0)),
            pl.BlockSpec((1, _CO), lambda b, t: (0, 0)),
            pl.BlockSpec((1, _CO), lambda b, t: (0, 0)),
            pl.BlockSpec((_CO, _CO), lambda b, t: (0, 0)),
            pl.BlockSpec((1, _CO), lambda b, t: (0, 0)),
        ],
        out_specs=[
            pl.BlockSpec((1, _TP, _CO), lambda b, t: (b, t, 0)),
            pl.BlockSpec((1, 8, _CO), lambda b, t: (b, 0, 0)),
        ],
        out_shape=[
            jax.ShapeDtypeStruct((_B, _NK, _CO), jnp.float32),
            jax.ShapeDtypeStruct((_B, 8, _CO), jnp.float32),
        ],
        compiler_params=_CP,
    )(rows, hft, st0, g0, be0, w1t, b1r)


def _mlp3_body(h1_ref, st_ref, g_ref, be_ref, out_ref):
    scale, shift = _fold(st_ref, g_ref, be_ref)
    h = h1_ref[0]
    a = _leaky(h * scale + shift)
    a3 = a.reshape(_TN2, _K, _CO)
    m = jnp.max(a3, axis=1)
    out_ref[0] = m.T


def _mlp3(h1, st1, g1, be1):
    grid = (_B, _N // _TN2)
    return pl.pallas_call(
        _mlp3_body,
        grid=grid,
        in_specs=[
            pl.BlockSpec((1, _TP, _CO), lambda b, t: (b, t, 0)),
            pl.BlockSpec((1, 8, _CO), lambda b, t: (b, 0, 0)),
            pl.BlockSpec((1, _CO), lambda b, t: (0, 0)),
            pl.BlockSpec((1, _CO), lambda b, t: (0, 0)),
        ],
        out_specs=pl.BlockSpec((1, _CO, _TN2), lambda b, t: (b, 0, t)),
        out_shape=jax.ShapeDtypeStruct((_B, _CO, _N), jnp.float32),
        compiler_params=_CP,
    )(h1, st1, g1, be1)


def kernel(xyz1, xyz2, feat1, feat2, W0, b0, g0, be0, W1, b1, g1, be1):
    xyz1t = jnp.transpose(xyz1, (0, 2, 1))
    xyz1p = jnp.concatenate(
        [xyz1, jnp.zeros((_B, 5, _N), jnp.float32)], axis=1)
    xyz2p = jnp.concatenate(
        [xyz2, jnp.zeros((_B, 5, _M), jnp.float32)], axis=1)
    wfa = W0[:, : _C1]
    wfb = jnp.zeros((_CO, 8), jnp.float32)
    wfb = wfb.at[:, :3].set(-W0[:, _C1 + _C2:])
    w2f = W0[:, _C1:_C1 + _C2]
    w2x = jnp.zeros((_CO, 8), jnp.float32)
    w2x = w2x.at[:, :3].set(W0[:, _C1 + _C2:])
    b0c = b0.reshape(_CO, 1)
    w1t = W1.T
    b1r = b1.reshape(1, _CO)
    g0r = g0.reshape(1, _CO)
    be0r = be0.reshape(1, _CO)
    g1r = g1.reshape(1, _CO)
    be1r = be1.reshape(1, _CO)

    fidx, tab, hft = _topk(xyz1t, xyz1p, xyz2p, feat2, feat1,
                           wfa, wfb, w2f, w2x, b0c)
    table = tab.reshape(_B * _M, _D)
    idx3 = fidx.reshape(32, (_B * _NK) // (32 * 128), 128)

    rows = _sc_gather(table, idx3).reshape(_B, _NK, _D)

    st0 = _mlp1(rows, hft)
    h1, st1 = _mlp2(rows, hft, st0, g0r, be0r, w1t, b1r)
    return _mlp3(h1, st1, g1r, be1r)


# speedup vs baseline: 14.5745x; 1.0147x over previous
import functools

import jax
import jax.numpy as jnp
from jax import lax
from jax.experimental import pallas as pl
from jax.experimental.pallas import tpu as pltpu
from jax.experimental.pallas import tpu_sc as plsc

_B = 2
_N = 2048
_M = 2048
_K = 16
_C1 = 64
_C2 = 64
_CO = 64
_D = 128
_DO = 64
_TN = 256
_TN2 = 256
_TP = _TN2 * _K
_NK = _N * _K
_EPS = 1e-5
_QB = (1 << 19) - 1
_CP = pltpu.CompilerParams(dimension_semantics=("parallel", "arbitrary"))


def _topk_body(x1t_ref, x1_ref, x2_ref, f2_ref, f1_ref,
               wfa_ref, wfb_ref, w2f_ref, w2x_ref, b0_ref,
               idx_ref, tab_ref, hf_ref):
    b = pl.program_id(0)
    t = pl.program_id(1)

    @pl.when(t == 0)
    def _():
        u = (
            jnp.dot(w2f_ref[...], f2_ref[0], preferred_element_type=jnp.float32)
            + jnp.dot(w2x_ref[...], x2_ref[0], preferred_element_type=jnp.float32)
        )
        tab_ref[0, :, 0:_DO] = u.T
        tab_ref[0, :, _DO:] = jnp.zeros((_M, _D - _DO), jnp.float32)

    hf = (
        jnp.dot(wfa_ref[...], f1_ref[0], preferred_element_type=jnp.float32)
        + jnp.dot(wfb_ref[...], x1_ref[0], preferred_element_type=jnp.float32)
        + b0_ref[...]
    )
    hf_ref[0] = hf

    x1t = x1t_ref[0]
    x2 = x2_ref[0]
    d0 = x1t[:, 0:1] - x2[0:1, :]
    d1 = x1t[:, 1:2] - x2[1:2, :]
    d2c = x1t[:, 2:3] - x2[2:3, :]
    d2 = d0 * d0 + d1 * d1 + d2c * d2c

    cm = jnp.min(d2.reshape(_TN, 16, 128), axis=2)
    hi = jnp.max(cm, axis=1, keepdims=True)
    scale = jnp.float32(_QB) / jnp.maximum(hi, jnp.float32(1e-37))
    dq = jnp.minimum(d2 * scale, jnp.float32(_QB))
    di = dq.astype(jnp.int32)
    iota = lax.broadcasted_iota(jnp.int32, (_TN, _M), 1)
    keys = lax.bitcast_convert_type(
        ((di << 11) | iota) + jnp.int32(0x00800000), jnp.float32)

    big = lax.bitcast_convert_type(jnp.int32(0x7F000000), jnp.float32)
    cols = []
    for _ in range(_K):
        mn = jnp.min(keys, axis=1, keepdims=True)
        cols.append(mn)
        keys = jnp.where(keys == mn, big, keys)
    idx = lax.bitcast_convert_type(
        jnp.concatenate(cols, axis=1), jnp.int32) & jnp.int32(0x7FF)
    idx_ref[0] = idx + b * _M


def _topk(xyz1t, xyz1, xyz2p, feat2, feat1, wfa, wfb, w2f, w2x, b0c):
    grid = (_B, _N // _TN)
    return pl.pallas_call(
        _topk_body,
        grid=grid,
        in_specs=[
            pl.BlockSpec((1, _TN, 3), lambda b, t: (b, t, 0)),
            pl.BlockSpec((1, 8, _TN), lambda b, t: (b, 0, t)),
            pl.BlockSpec((1, 8, _M), lambda b, t: (b, 0, 0)),
            pl.BlockSpec((1, _C2, _M), lambda b, t: (b, 0, 0)),
            pl.BlockSpec((1, _C1, _TN), lambda b, t: (b, 0, t)),
            pl.BlockSpec((_CO, _C1), lambda b, t: (0, 0)),
            pl.BlockSpec((_CO, 8), lambda b, t: (0, 0)),
            pl.BlockSpec((_CO, _C2), lambda b, t: (0, 0)),
            pl.BlockSpec((_CO, 8), lambda b, t: (0, 0)),
            pl.BlockSpec((_CO, 1), lambda b, t: (0, 0)),
        ],
        out_specs=[
            pl.BlockSpec((1, _TN, _K), lambda b, t: (b, t, 0)),
            pl.BlockSpec((1, _M, _D), lambda b, t: (b, 0, 0)),
            pl.BlockSpec((1, _CO, _TN), lambda b, t: (b, 0, t)),
        ],
        out_shape=[
            jax.ShapeDtypeStruct((_B, _N, _K), jnp.int32),
            jax.ShapeDtypeStruct((_B, _M, _D), jnp.float32),
            jax.ShapeDtypeStruct((_B, _CO, _N), jnp.float32),
        ],
        compiler_params=_CP,
    )(xyz1t, xyz1, xyz2p, feat2, feat1, wfa, wfb, w2f, w2x, b0c)


def _sc_gather(table, idx3):
    info = plsc.get_sparse_core_info()
    nw = info.num_cores * info.num_subcores
    btot = _B * _NK
    b_per_w = btot // nw
    ch_sz = 128
    n_ch = b_per_w // ch_sz
    mesh = plsc.VectorSubcoreMesh(core_axis_name="c", subcore_axis_name="s")

    @functools.partial(
        pl.kernel,
        mesh=mesh,
        out_type=jax.ShapeDtypeStruct((btot, _D), jnp.float32),
        scratch_types=[
            pltpu.VMEM((n_ch, ch_sz), jnp.int32),
            pltpu.VMEM((ch_sz, _D), jnp.float32),
            pltpu.SemaphoreType.DMA,
        ],
    )
    def gk(table_hbm, idx_hbm, out_hbm, idx_v, rows_v, sem):
        wid = lax.axis_index("s") * info.num_cores + lax.axis_index("c")
        base = wid * b_per_w
        pltpu.sync_copy(idx_hbm.at[wid], idx_v)
        for ch in range(n_ch):
            pltpu.async_copy(table_hbm.at[idx_v.at[ch]], rows_v, sem).wait()
            pltpu.sync_copy(rows_v, out_hbm.at[pl.ds(base + ch * ch_sz, ch_sz)])

    return gk(table, idx3)


def _leaky(x):
    return jnp.where(x >= 0, x, 0.1 * x)


def _fold(st_ref, g_ref, be_ref):
    sq = st_ref[0, 0:2, :]
    ri = lax.broadcasted_iota(jnp.int32, (_CO, _CO), 0) >> 4
    ci = lax.broadcasted_iota(jnp.int32, (_CO, _CO), 1) >> 4
    G = (ri == ci).astype(jnp.float32)
    sqg = jnp.dot(sq, G, preferred_element_type=jnp.float32)
    count = jnp.float32(16 * _N * _K)
    mean = sqg[0:1, :] / count
    var = sqg[1:2, :] / count - mean * mean
    inv = 1.0 / jnp.sqrt(var + _EPS)
    scale = g_ref[...] * inv
    shift = be_ref[...] - mean * scale
    return scale, shift


def _h0_tile(rows_ref, hf_ref):
    g = rows_ref[0][:, 0:_DO]
    hf = hf_ref[0].T
    h = g.reshape(_TN2, _K, _CO) + hf[:, None, :]
    return h.reshape(_TP, _CO)


def _mlp1_body(rows_ref, hf_ref, st_ref):
    t = pl.program_id(1)
    h2 = _h0_tile(rows_ref, hf_ref)
    s = jnp.sum(h2, axis=0, keepdims=True)
    q = jnp.sum(h2 * h2, axis=0, keepdims=True)
    st = jnp.concatenate([s, q, jnp.zeros((6, _CO), jnp.float32)], axis=0)

    @pl.when(t == 0)
    def _():
        st_ref[0] = st

    @pl.when(t > 0)
    def _():
        st_ref[0] += st


def _mlp1(rows, hft):
    grid = (_B, _N // _TN2)
    return pl.pallas_call(
        _mlp1_body,
        grid=grid,
        in_specs=[
            pl.BlockSpec((1, _TP, _D), lambda b, t: (b, t, 0)),
            pl.BlockSpec((1, _CO, _TN2), lambda b, t: (b, 0, t)),
        ],
        out_specs=pl.BlockSpec((1, 8, _CO), lambda b, t: (b, 0, 0)),
        out_shape=jax.ShapeDtypeStruct((_B, 8, _CO), jnp.float32),
        compiler_params=_CP,
    )(rows, hft)


def _mlp2_body(rows_ref, hf_ref, st_ref, g_ref, be_ref, w1_ref, b1_ref,
               so_ref):
    t = pl.program_id(1)
    scale, shift = _fold(st_ref, g_ref, be_ref)
    h = _h0_tile(rows_ref, hf_ref)
    a = _leaky(h * scale + shift)
    h1 = jnp.dot(a, w1_ref[...], preferred_element_type=jnp.float32) + b1_ref[...]
    s = jnp.sum(h1, axis=0, keepdims=True)
    q = jnp.sum(h1 * h1, axis=0, keepdims=True)
    st = jnp.concatenate([s, q, jnp.zeros((6, _CO), jnp.float32)], axis=0)

    @pl.when(t == 0)
    def _():
        so_ref[0] = st

    @pl.when(t > 0)
    def _():
        so_ref[0] += st


def _mlp2(rows, hft, st0, g0, be0, w1t, b1r):
    grid = (_B, _N // _TN2)
    return pl.pallas_call(
        _mlp2_body,
        grid=grid,
        in_specs=[
            pl.BlockSpec((1, _TP, _D), lambda b, t: (b, t, 0)),
            pl.BlockSpec((1, _CO, _TN2), lambda b, t: (b, 0, t)),
            pl.BlockSpec((1, 8, _CO), lambda b, t: (b, 0, 0)),
            pl.BlockSpec((1, _CO), lambda b, t: (0, 0)),
            pl.BlockSpec((1, _CO), lambda b, t: (0, 0)),
            pl.BlockSpec((_CO, _CO), lambda b, t: (0, 0)),
            pl.BlockSpec((1, _CO), lambda b, t: (0, 0)),
        ],
        out_specs=pl.BlockSpec((1, 8, _CO), lambda b, t: (b, 0, 0)),
        out_shape=jax.ShapeDtypeStruct((_B, 8, _CO), jnp.float32),
        compiler_params=_CP,
    )(rows, hft, st0, g0, be0, w1t, b1r)


def _mlp3_body(rows_ref, hf_ref, st0_ref, g0_ref, be0_ref, w1_ref, b1_ref,
               st1_ref, g1_ref, be1_ref, out_ref):
    scale0, shift0 = _fold(st0_ref, g0_ref, be0_ref)
    scale1, shift1 = _fold(st1_ref, g1_ref, be1_ref)
    h = _h0_tile(rows_ref, hf_ref)
    a = _leaky(h * scale0 + shift0)
    h1 = jnp.dot(a, w1_ref[...], preferred_element_type=jnp.float32) + b1_ref[...]
    a1 = _leaky(h1 * scale1 + shift1)
    a3 = a1.reshape(_TN2, _K, _CO)
    m = jnp.max(a3, axis=1)
    out_ref[0] = m.T


def _mlp3(rows, hft, st0, g0, be0, w1t, b1r, st1, g1, be1):
    grid = (_B, _N // _TN2)
    return pl.pallas_call(
        _mlp3_body,
        grid=grid,
        in_specs=[
            pl.BlockSpec((1, _TP, _D), lambda b, t: (b, t, 0)),
            pl.BlockSpec((1, _CO, _TN2), lambda b, t: (b, 0, t)),
            pl.BlockSpec((1, 8, _CO), lambda b, t: (b, 0, 0)),
            pl.BlockSpec((1, _CO), lambda b, t: (0, 0)),
            pl.BlockSpec((1, _CO), lambda b, t: (0, 0)),
            pl.BlockSpec((_CO, _CO), lambda b, t: (0, 0)),
            pl.BlockSpec((1, _CO), lambda b, t: (0, 0)),
            pl.BlockSpec((1, 8, _CO), lambda b, t: (b, 0, 0)),
            pl.BlockSpec((1, _CO), lambda b, t: (0, 0)),
            pl.BlockSpec((1, _CO), lambda b, t: (0, 0)),
        ],
        out_specs=pl.BlockSpec((1, _CO, _TN2), lambda b, t: (b, 0, t)),
        out_shape=jax.ShapeDtypeStruct((_B, _CO, _N), jnp.float32),
        compiler_params=_CP,
    )(rows, hft, st0, g0, be0, w1t, b1r, st1, g1, be1)


def kernel(xyz1, xyz2, feat1, feat2, W0, b0, g0, be0, W1, b1, g1, be1):
    xyz1t = jnp.transpose(xyz1, (0, 2, 1))
    xyz1p = jnp.concatenate(
        [xyz1, jnp.zeros((_B, 5, _N), jnp.float32)], axis=1)
    xyz2p = jnp.concatenate(
        [xyz2, jnp.zeros((_B, 5, _M), jnp.float32)], axis=1)
    wfa = W0[:, : _C1]
    wfb = jnp.zeros((_CO, 8), jnp.float32)
    wfb = wfb.at[:, :3].set(-W0[:, _C1 + _C2:])
    w2f = W0[:, _C1:_C1 + _C2]
    w2x = jnp.zeros((_CO, 8), jnp.float32)
    w2x = w2x.at[:, :3].set(W0[:, _C1 + _C2:])
    b0c = b0.reshape(_CO, 1)
    w1t = W1.T
    b1r = b1.reshape(1, _CO)
    g0r = g0.reshape(1, _CO)
    be0r = be0.reshape(1, _CO)
    g1r = g1.reshape(1, _CO)
    be1r = be1.reshape(1, _CO)

    fidx, tab, hft = _topk(xyz1t, xyz1p, xyz2p, feat2, feat1,
                           wfa, wfb, w2f, w2x, b0c)
    table = tab.reshape(_B * _M, _D)
    idx3 = fidx.reshape(32, (_B * _NK) // (32 * 128), 128)

    rows = _sc_gather(table, idx3).reshape(_B, _NK, _D)

    st0 = _mlp1(rows, hft)
    st1 = _mlp2(rows, hft, st0, g0r, be0r, w1t, b1r)
    return _mlp3(rows, hft, st0, g0r, be0r, w1t, b1r,
                 st1, g1r, be1r)
